# baseline (speedup 1.0000x reference)
import functools

import jax
import jax.numpy as jnp
from jax import lax
from jax.experimental import pallas as pl
from jax.experimental.pallas import tpu as pltpu
from jax.experimental.pallas import tpu_sc as plsc

NUM_OBJ = 151
NUM_REL = 51
BATCH = 16384

NC, NS, L = 2, 16, 16
NW = NC * NS
B_PER_W = BATCH // NW
D_PAD = 128
G = 128
N_CHUNK = B_PER_W // G


@jax.jit
def _sc_gather(l0, l1, table_pad):
    mesh = plsc.VectorSubcoreMesh(core_axis_name="c", subcore_axis_name="s")

    @functools.partial(
        pl.kernel,
        mesh=mesh,
        out_type=jax.ShapeDtypeStruct((BATCH, D_PAD), jnp.float32),
        scratch_types=[
            pltpu.VMEM((B_PER_W,), jnp.int32),
            pltpu.VMEM((B_PER_W,), jnp.int32),
            pltpu.VMEM((B_PER_W,), jnp.int32),
            pltpu.VMEM((B_PER_W, D_PAD), jnp.float32),
            pltpu.SemaphoreType.DMA,
        ],
    )
    def k(l0_hbm, l1_hbm, table_hbm, out_hbm, l0_v, l1_v, idx_v, rows_v, sem):
        wid = lax.axis_index("s") * NC + lax.axis_index("c")
        base = wid * B_PER_W
        pltpu.sync_copy(l0_hbm.at[pl.ds(base, B_PER_W)], l0_v)
        pltpu.sync_copy(l1_hbm.at[pl.ds(base, B_PER_W)], l1_v)

        @pl.loop(0, B_PER_W, step=L)
        def _(c):
            sl = pl.ds(c, L)
            idx_v.at[sl][...] = l0_v.at[sl][...] * NUM_OBJ + l1_v.at[sl][...]

        for j in range(N_CHUNK):
            pltpu.async_copy(
                table_hbm.at[idx_v.at[pl.ds(j * G, G)]],
                rows_v.at[pl.ds(j * G, G)],
                sem,
            )
        for j in range(N_CHUNK):
            pltpu.make_async_copy(
                table_hbm.at[idx_v.at[pl.ds(j * G, G)]],
                rows_v.at[pl.ds(j * G, G)],
                sem,
            ).wait()

        pltpu.sync_copy(rows_v, out_hbm.at[pl.ds(base, B_PER_W)])

    return k(l0, l1, table_pad)


def kernel(labels, table):
    l0 = labels[:, 0].astype(jnp.int32)
    l1 = labels[:, 1].astype(jnp.int32)
    table_pad = jnp.pad(table, ((0, 0), (0, D_PAD - NUM_REL)))
    out_pad = _sc_gather(l0, l1, table_pad)
    return out_pad[:, :NUM_REL]
